# baseline (speedup 1.0000x reference)
import functools

import jax
import jax.numpy as jnp
from jax import lax
from jax.experimental import pallas as pl
from jax.experimental.pallas import tpu as pltpu
from jax.experimental.pallas import tpu_sc as plsc

B = 4096
L = 200
DIM = 64
MAXLEN = 256
NC = 2
NS = 16
NW = NC * NS
BW = B // NW
NBUF = 4
SCALE = 0.7071067811865476


def _bcast_lane(vec, i):
    idx = jnp.full((16, 1), i, jnp.int32)
    return lax.gather(
        vec,
        idx,
        dimension_numbers=lax.GatherDimensionNumbers(
            offset_dims=(), collapsed_slice_dims=(0,), start_index_map=(0,)
        ),
        slice_sizes=(1,),
        mode=lax.GatherScatterMode.PROMISE_IN_BOUNDS,
    )


def _sc_body(
    labelsT_hbm,
    table_hbm,
    pos_hbm,
    out_hbm,
    labT,
    pos_v,
    e0, e1, e2, e3,
    b0, b1, b2, b3,
    g0, g1, g2, g3,
    o0, o1, o2, o3,
):
    wid = lax.axis_index("s") * NC + lax.axis_index("c")
    ebufs = [e0, e1, e2, e3]
    obufs = [b0, b1, b2, b3]
    gsems = [g0, g1, g2, g3]
    osems = [o0, o1, o2, o3]

    pltpu.sync_copy(labelsT_hbm.at[:, pl.ds(BW * wid, BW)], labT)
    pltpu.sync_copy(pos_hbm, pos_v)

    def fire_gather(c, buf):
        pltpu.async_copy(table_hbm.at[labT.at[c]], ebufs[buf], gsems[buf])

    def wait_gather(buf):
        pltpu.make_async_copy(
            table_hbm.at[labT.at[0]], ebufs[buf], gsems[buf]
        ).wait()

    def fire_out(c, buf):
        pltpu.async_copy(
            obufs[buf],
            out_hbm.at[pl.ds(8 * c, 8), pl.ds(wid, 1)],
            osems[buf],
        )

    def wait_out(buf):
        pltpu.make_async_copy(
            obufs[buf],
            out_hbm.at[pl.ds(0, 8), pl.ds(0, 1)],
            osems[buf],
        ).wait()

    iota16 = lax.iota(jnp.int32, 16)

    def compute(c, buf):
        E = ebufs[buf]
        ob = obufs[buf]

        def bg_body(bg, carry):
            lab16 = labT[c, pl.ds(16 * bg, 16)]
            m16 = lax.sign(lab16).astype(jnp.float32)
            rows16 = iota16 + 16 * bg
            for fg in range(DIM // 16):
                p16 = pos_v[c + 1, pl.ds(16 * fg, 16)]
                for ff in range(16):
                    f = 16 * fg + ff
                    e16 = plsc.load_gather(
                        E, [rows16, jnp.full((16,), f, jnp.int32)]
                    )
                    pf = _bcast_lane(p16, ff)
                    o = (e16 + pf * m16) * jnp.float32(SCALE)
                    ob[f // 8, 0, f % 8, pl.ds(16 * bg, 16)] = o
            return carry

        lax.fori_loop(0, BW // 16, bg_body, 0)

    fire_gather(0, 0)
    fire_gather(1, 1)

    def outer(k, carry):
        for j in range(NBUF):
            c = NBUF * k + j

            @pl.when(c + 2 < L)
            def _():
                fire_gather(c + 2, (j + 2) % NBUF)

            wait_gather(j)

            @pl.when(k > 0)
            def _():
                wait_out(j)

            compute(c, j)
            fire_out(c, j)
        return carry

    lax.fori_loop(0, L // NBUF, outer, 0)

    for j in range(NBUF):
        wait_out(j)


@functools.partial(jax.jit, static_argnames=())
def _run(labelsT, table, pos_table):
    mesh = plsc.VectorSubcoreMesh(
        core_axis_name="c", subcore_axis_name="s", num_cores=NC, num_subcores=NS
    )
    f = pl.kernel(
        _sc_body,
        out_type=jax.ShapeDtypeStruct((L * 8, NW, 8, 128), jnp.float32),
        mesh=mesh,
        compiler_params=pltpu.CompilerParams(
            use_tc_tiling_on_sc=False, needs_layout_passes=False
        ),
        scratch_types=(
            [
                pltpu.VMEM((L, 128), jnp.int32),
                pltpu.VMEM((MAXLEN, DIM), jnp.float32),
            ]
            + [pltpu.VMEM((BW, DIM), jnp.float32) for _ in range(NBUF)]
            + [pltpu.VMEM((8, 1, 8, 128), jnp.float32) for _ in range(NBUF)]
            + [pltpu.SemaphoreType.DMA for _ in range(2 * NBUF)]
        ),
    )
    return f(labelsT, table, pos_table)


def kernel(labels, label_table, pos_table):
    labelsT = labels.astype(jnp.int32).T
    out4 = _run(labelsT, label_table, pos_table)
    out = (
        out4.reshape(L, 8, NW, 8, 128)
        .transpose(2, 4, 0, 1, 3)
        .reshape(B, L, DIM)
    )
    return out


# speedup vs baseline: 1.1299x; 1.1299x over previous
import functools

import jax
import jax.numpy as jnp
from jax import lax
from jax.experimental import pallas as pl
from jax.experimental.pallas import tpu as pltpu
from jax.experimental.pallas import tpu_sc as plsc

B = 4096
L = 200
DIM = 64
MAXLEN = 256
NC = 2
NS = 16
NW = NC * NS
BW = B // NW
NBUF = 4
SCALE = 0.7071067811865476


def _bcast_lane(vec, i):
    idx = jnp.full((16, 1), i, jnp.int32)
    return lax.gather(
        vec,
        idx,
        dimension_numbers=lax.GatherDimensionNumbers(
            offset_dims=(), collapsed_slice_dims=(0,), start_index_map=(0,)
        ),
        slice_sizes=(1,),
        mode=lax.GatherScatterMode.PROMISE_IN_BOUNDS,
    )


def _sc_body(
    labelsT_hbm,
    table_hbm,
    pos_hbm,
    out_hbm,
    labT,
    pos_v,
    e0, e1, e2, e3,
    b0, b1, b2, b3,
    g0, g1, g2, g3,
    o0, o1, o2, o3,
):
    wid = lax.axis_index("s") * NC + lax.axis_index("c")
    ebufs = [e0, e1, e2, e3]
    obufs = [b0, b1, b2, b3]
    gsems = [g0, g1, g2, g3]
    osems = [o0, o1, o2, o3]

    pltpu.sync_copy(labelsT_hbm.at[:, pl.ds(BW * wid, BW)], labT)
    pltpu.sync_copy(pos_hbm, pos_v)

    def fire_gather(c, buf):
        pltpu.async_copy(table_hbm.at[labT.at[c]], ebufs[buf], gsems[buf])

    def wait_gather(buf):
        pltpu.make_async_copy(
            table_hbm.at[labT.at[0]], ebufs[buf], gsems[buf]
        ).wait()

    def fire_out(c, buf):
        for tf in range(8):
            pltpu.async_copy(
                obufs[buf].at[pl.ds(1024 * tf, 1024)],
                out_hbm.at[pl.ds(((8 * c + tf) * NW + wid) * 1024, 1024)],
                osems[buf],
            )

    def wait_out(buf):
        for tf in range(8):
            pltpu.make_async_copy(
                obufs[buf].at[pl.ds(1024 * tf, 1024)],
                out_hbm.at[pl.ds(1024 * tf, 1024)],
                osems[buf],
            ).wait()

    iota16 = lax.iota(jnp.int32, 16)
    scat = [iota16 * 128 + 16 * 128 * j for j in range(DIM // 16)]

    def compute(c, buf):
        E = ebufs[buf]
        ob = obufs[buf]
        sp = [
            pos_v[c + 1, pl.ds(16 * j, 16)] * jnp.float32(SCALE)
            for j in range(DIM // 16)
        ]

        def bg_body(bg, carry):
            lab16 = labT[c, pl.ds(16 * bg, 16)]
            m16 = lax.sign(lab16).astype(jnp.float32)
            for i in range(16):
                r = 16 * bg + i
                m = _bcast_lane(m16, i)
                rbc = jnp.full((16,), 1, jnp.int32) * r
                for j in range(DIM // 16):
                    e = E[r, pl.ds(16 * j, 16)]
                    o = e * jnp.float32(SCALE) + sp[j] * m
                    plsc.store_scatter(ob, [scat[j] + rbc], o)
            return carry

        lax.fori_loop(0, BW // 16, bg_body, 0)

    fire_gather(0, 0)
    fire_gather(1, 1)

    def outer(k, carry):
        for j in range(NBUF):
            c = NBUF * k + j

            @pl.when(c + 2 < L)
            def _():
                fire_gather(c + 2, (j + 2) % NBUF)

            wait_gather(j)

            @pl.when(k > 0)
            def _():
                wait_out(j)

            compute(c, j)
            fire_out(c, j)
        return carry

    lax.fori_loop(0, L // NBUF, outer, 0)

    for j in range(NBUF):
        wait_out(j)


@functools.partial(jax.jit, static_argnames=())
def _run(labelsT, table, pos_table):
    mesh = plsc.VectorSubcoreMesh(
        core_axis_name="c", subcore_axis_name="s", num_cores=NC, num_subcores=NS
    )
    f = pl.kernel(
        _sc_body,
        out_type=jax.ShapeDtypeStruct((L * 8 * NW * 8 * 128,), jnp.float32),
        mesh=mesh,
        compiler_params=pltpu.CompilerParams(
            use_tc_tiling_on_sc=False, needs_layout_passes=False
        ),
        scratch_types=(
            [
                pltpu.VMEM((L, 128), jnp.int32),
                pltpu.VMEM((MAXLEN, DIM), jnp.float32),
            ]
            + [pltpu.VMEM((BW, DIM), jnp.float32) for _ in range(NBUF)]
            + [pltpu.VMEM((8192,), jnp.float32) for _ in range(NBUF)]
            + [pltpu.SemaphoreType.DMA for _ in range(2 * NBUF)]
        ),
    )
    return f(labelsT, table, pos_table)


def kernel(labels, label_table, pos_table):
    labelsT = labels.astype(jnp.int32).T
    out4 = _run(labelsT, label_table, pos_table)
    out = (
        out4.reshape(L, 8, NW, 8, 128)
        .transpose(2, 4, 0, 1, 3)
        .reshape(B, L, DIM)
    )
    return out


# speedup vs baseline: 1.6774x; 1.4846x over previous
import functools

import jax
import jax.numpy as jnp
from jax import lax
from jax.experimental import pallas as pl
from jax.experimental.pallas import tpu as pltpu
from jax.experimental.pallas import tpu_sc as plsc

B = 4096
L = 200
DIM = 64
MAXLEN = 256
NC = 2
NS = 16
NW = NC * NS
BW = B // NW
NBUF = 4
SCALE = 0.7071067811865476


def _bcast_lane(vec, i):
    idx = jnp.full((16, 1), i, jnp.int32)
    return lax.gather(
        vec,
        idx,
        dimension_numbers=lax.GatherDimensionNumbers(
            offset_dims=(), collapsed_slice_dims=(0,), start_index_map=(0,)
        ),
        slice_sizes=(1,),
        mode=lax.GatherScatterMode.PROMISE_IN_BOUNDS,
    )


def _sc_body(
    labelsT_hbm,
    table_hbm,
    pos_hbm,
    out_hbm,
    labT,
    pos_v,
    e0, e1, e2, e3,
    b0, b1, b2, b3,
    g0, g1, g2, g3,
    o0, o1, o2, o3,
):
    wid = lax.axis_index("s") * NC + lax.axis_index("c")
    ebufs = [e0, e1, e2, e3]
    obufs = [b0, b1, b2, b3]
    gsems = [g0, g1, g2, g3]
    osems = [o0, o1, o2, o3]

    pltpu.sync_copy(labelsT_hbm.at[:, pl.ds(BW * wid, BW)], labT)
    pltpu.sync_copy(pos_hbm, pos_v)

    def fire_gather(c, buf):
        pltpu.async_copy(table_hbm.at[labT.at[c]], ebufs[buf], gsems[buf])

    def wait_gather(buf):
        pltpu.make_async_copy(
            table_hbm.at[labT.at[0]], ebufs[buf], gsems[buf]
        ).wait()

    def fire_out(c, buf):
        for tf in range(8):
            pltpu.async_copy(
                obufs[buf].at[pl.ds(8 * tf, 8), pl.ds(0, 128)],
                out_hbm.at[(8 * c + tf) * NW + wid],
                osems[buf],
            )

    def wait_out(buf):
        for tf in range(8):
            pltpu.make_async_copy(
                obufs[buf].at[pl.ds(8 * tf, 8), pl.ds(0, 128)],
                out_hbm.at[0],
                osems[buf],
            ).wait()

    iota16 = lax.iota(jnp.int32, 16)
    scat = [iota16 + 16 * j for j in range(DIM // 16)]

    def compute(c, buf):
        E = ebufs[buf]
        ob2 = obufs[buf]
        sp = [
            pos_v[c + 1, pl.ds(16 * j, 16)] * jnp.float32(SCALE)
            for j in range(DIM // 16)
        ]

        def bg_body(bg, carry):
            lab16 = labT[c, pl.ds(16 * bg, 16)]
            m16 = lax.sign(lab16).astype(jnp.float32)
            for i in range(16):
                r = 16 * bg + i
                m = _bcast_lane(m16, i)
                rbc = jnp.full((16,), 1, jnp.int32) * r
                for j in range(DIM // 16):
                    e = E[r, pl.ds(16 * j, 16)]
                    o = e * jnp.float32(SCALE) + sp[j] * m
                    plsc.store_scatter(ob2, [scat[j], rbc], o)
            return carry

        lax.fori_loop(0, BW // 16, bg_body, 0)

    fire_gather(0, 0)
    fire_gather(1, 1)

    def outer(k, carry):
        for j in range(NBUF):
            c = NBUF * k + j

            @pl.when(c + 2 < L)
            def _():
                fire_gather(c + 2, (j + 2) % NBUF)

            wait_gather(j)

            @pl.when(k > 0)
            def _():
                wait_out(j)

            compute(c, j)
            fire_out(c, j)
        return carry

    lax.fori_loop(0, L // NBUF, outer, 0)

    for j in range(NBUF):
        wait_out(j)


@functools.partial(jax.jit, static_argnames=())
def _run(labelsT, table, pos_table):
    mesh = plsc.VectorSubcoreMesh(
        core_axis_name="c", subcore_axis_name="s", num_cores=NC, num_subcores=NS
    )
    f = pl.kernel(
        _sc_body,
        out_type=jax.ShapeDtypeStruct((L * 8 * NW, 8, 128), jnp.float32),
        mesh=mesh,
        compiler_params=pltpu.CompilerParams(
            use_tc_tiling_on_sc=False, needs_layout_passes=False
        ),
        scratch_types=(
            [
                pltpu.VMEM((L, 128), jnp.int32),
                pltpu.VMEM((MAXLEN, DIM), jnp.float32),
            ]
            + [pltpu.VMEM((BW, DIM), jnp.float32) for _ in range(NBUF)]
            + [pltpu.VMEM((64, 129), jnp.float32) for _ in range(NBUF)]
            + [pltpu.SemaphoreType.DMA for _ in range(2 * NBUF)]
        ),
    )
    return f(labelsT, table, pos_table)


def kernel(labels, label_table, pos_table):
    labelsT = labels.astype(jnp.int32).T
    out4 = _run(labelsT, label_table, pos_table)
    out = (
        out4.reshape(L, 8, NW, 8, 128)
        .transpose(2, 4, 0, 1, 3)
        .reshape(B, L, DIM)
    )
    return out


# speedup vs baseline: 2.0876x; 1.2445x over previous
import functools

import jax
import jax.numpy as jnp
from jax import lax
from jax.experimental import pallas as pl
from jax.experimental.pallas import tpu as pltpu
from jax.experimental.pallas import tpu_sc as plsc

B = 4096
L = 200
DIM = 64
MAXLEN = 256
NC = 2
NS = 16
NW = NC * NS
BW = B // NW
NBUF = 4
SCALE = 0.7071067811865476


def _bcast_lane(vec, i):
    idx = jnp.full((16, 1), i, jnp.int32)
    return lax.gather(
        vec,
        idx,
        dimension_numbers=lax.GatherDimensionNumbers(
            offset_dims=(), collapsed_slice_dims=(0,), start_index_map=(0,)
        ),
        slice_sizes=(1,),
        mode=lax.GatherScatterMode.PROMISE_IN_BOUNDS,
    )


def _sc_body(
    labelsT_hbm,
    table_hbm,
    pos_hbm,
    out_hbm,
    labT,
    pos_v,
    e0, e1, e2, e3,
    b0, b1, b2, b3,
    g0, g1, g2, g3,
    o0, o1, o2, o3,
):
    wid = lax.axis_index("s") * NC + lax.axis_index("c")
    ebufs = [e0, e1, e2, e3]
    obufs = [b0, b1, b2, b3]
    gsems = [g0, g1, g2, g3]
    osems = [o0, o1, o2, o3]

    pltpu.sync_copy(labelsT_hbm.at[:, pl.ds(BW * wid, BW)], labT)
    pltpu.sync_copy(pos_hbm, pos_v)

    def fire_gather(c, buf):
        pltpu.async_copy(table_hbm.at[labT.at[c]], ebufs[buf], gsems[buf])

    def wait_gather(buf):
        pltpu.make_async_copy(
            table_hbm.at[labT.at[0]], ebufs[buf], gsems[buf]
        ).wait()

    def fire_out(c, buf):
        for tf in range(8):
            pltpu.async_copy(
                obufs[buf].at[pl.ds(8 * tf, 8), pl.ds(0, 128)],
                out_hbm.at[(8 * c + tf) * NW + wid],
                osems[buf],
            )

    def wait_out(buf):
        for tf in range(8):
            pltpu.make_async_copy(
                obufs[buf].at[pl.ds(8 * tf, 8), pl.ds(0, 128)],
                out_hbm.at[0],
                osems[buf],
            ).wait()

    iota16 = lax.iota(jnp.int32, 16)
    scat = [iota16 + 16 * j for j in range(DIM // 16)]

    def compute(c, buf):
        E = ebufs[buf]
        ob2 = obufs[buf]
        sp = [
            pos_v[c + 1, pl.ds(16 * j, 16)] * jnp.float32(SCALE)
            for j in range(DIM // 16)
        ]

        def bg_body(bg, carry):
            lab16 = labT[c, pl.ds(16 * bg, 16)]
            m16 = lax.sign(lab16).astype(jnp.float32)
            for i in range(16):
                r = 16 * bg + i
                m = _bcast_lane(m16, i)
                rbc = jnp.full((16,), 1, jnp.int32) * r
                es = [E[r, pl.ds(16 * j, 16)] for j in range(DIM // 16)]
                os_ = [
                    es[j] * jnp.float32(SCALE) + sp[j] * m
                    for j in range(DIM // 16)
                ]
                for j in range(DIM // 16):
                    plsc.store_scatter(ob2, [scat[j], rbc], os_[j])
            return carry

        lax.fori_loop(0, BW // 16, bg_body, 0)

    fire_gather(0, 0)
    fire_gather(1, 1)

    def outer(k, carry):
        for j in range(NBUF):
            c = NBUF * k + j

            @pl.when(c + 2 < L)
            def _():
                fire_gather(c + 2, (j + 2) % NBUF)

            wait_gather(j)

            @pl.when(k > 0)
            def _():
                wait_out(j)

            compute(c, j)
            fire_out(c, j)
        return carry

    lax.fori_loop(0, L // NBUF, outer, 0)

    for j in range(NBUF):
        wait_out(j)


@functools.partial(jax.jit, static_argnames=())
def _run(labelsT, table, pos_table):
    mesh = plsc.VectorSubcoreMesh(
        core_axis_name="c", subcore_axis_name="s", num_cores=NC, num_subcores=NS
    )
    f = pl.kernel(
        _sc_body,
        out_type=jax.ShapeDtypeStruct((L * 8 * NW, 8, 128), jnp.float32),
        mesh=mesh,
        compiler_params=pltpu.CompilerParams(
            use_tc_tiling_on_sc=False, needs_layout_passes=False
        ),
        scratch_types=(
            [
                pltpu.VMEM((L, 128), jnp.int32),
                pltpu.VMEM((MAXLEN, DIM), jnp.float32),
            ]
            + [pltpu.VMEM((BW, DIM), jnp.float32) for _ in range(NBUF)]
            + [pltpu.VMEM((64, 129), jnp.float32) for _ in range(NBUF)]
            + [pltpu.SemaphoreType.DMA for _ in range(2 * NBUF)]
        ),
    )
    return f(labelsT, table, pos_table)


def kernel(labels, label_table, pos_table):
    labelsT = labels.astype(jnp.int32).T
    out4 = _run(labelsT, label_table, pos_table)
    out = (
        out4.reshape(L, 8, NW, 8, 128)
        .transpose(2, 4, 0, 1, 3)
        .reshape(B, L, DIM)
    )
    return out


# speedup vs baseline: 2.2884x; 1.0962x over previous
import functools

import jax
import jax.numpy as jnp
from jax import lax
from jax.experimental import pallas as pl
from jax.experimental.pallas import tpu as pltpu
from jax.experimental.pallas import tpu_sc as plsc

B = 4096
L = 200
DIM = 64
MAXLEN = 256
NC = 2
NS = 16
NW = NC * NS
BW = B // NW
NBUF = 4
SCALE = 0.7071067811865476


def _bcast_lane(vec, i):
    idx = jnp.full((16, 1), i, jnp.int32)
    return lax.gather(
        vec,
        idx,
        dimension_numbers=lax.GatherDimensionNumbers(
            offset_dims=(), collapsed_slice_dims=(0,), start_index_map=(0,)
        ),
        slice_sizes=(1,),
        mode=lax.GatherScatterMode.PROMISE_IN_BOUNDS,
    )


def _sc_body(
    labelsT_hbm,
    table_hbm,
    pos_hbm,
    out_hbm,
    labT,
    pos_v,
    e0, e1, e2, e3,
    b0, b1, b2, b3,
    g0, g1, g2, g3,
    o0, o1, o2, o3,
):
    wid = lax.axis_index("s") * NC + lax.axis_index("c")
    ebufs = [e0, e1, e2, e3]
    obufs = [b0, b1, b2, b3]
    gsems = [g0, g1, g2, g3]
    osems = [o0, o1, o2, o3]

    pltpu.sync_copy(labelsT_hbm.at[:, pl.ds(BW * wid, BW)], labT)
    pltpu.sync_copy(pos_hbm, pos_v)

    def fire_gather(c, buf):
        pltpu.async_copy(table_hbm.at[labT.at[c]], ebufs[buf], gsems[buf])

    def wait_gather(buf):
        pltpu.make_async_copy(
            table_hbm.at[labT.at[0]], ebufs[buf], gsems[buf]
        ).wait()

    def fire_out(c, buf):
        for tf in range(8):
            pltpu.async_copy(
                obufs[buf].at[pl.ds(8 * tf, 8), pl.ds(0, 128)],
                out_hbm.at[(8 * c + tf) * NW + wid],
                osems[buf],
            )

    def wait_out(buf):
        for tf in range(8):
            pltpu.make_async_copy(
                obufs[buf].at[pl.ds(8 * tf, 8), pl.ds(0, 128)],
                out_hbm.at[0],
                osems[buf],
            ).wait()

    iota16 = lax.iota(jnp.int32, 16)
    scat = [iota16 + 16 * j for j in range(DIM // 16)]

    def compute(c, buf):
        E = ebufs[buf]
        ob2 = obufs[buf]
        sp = [
            pos_v[c + 1, pl.ds(16 * j, 16)] * jnp.float32(SCALE)
            for j in range(DIM // 16)
        ]

        def bg_body(bg, carry):
            lab16 = labT[c, pl.ds(16 * bg, 16)]
            m16 = lax.sign(lab16).astype(jnp.float32)
            for i in range(0, 16, 2):
                rr = [16 * bg + i, 16 * bg + i + 1]
                ms = [_bcast_lane(m16, i), _bcast_lane(m16, i + 1)]
                rbcs = [jnp.full((16,), 1, jnp.int32) * r for r in rr]
                es = [
                    E[r, pl.ds(16 * j, 16)]
                    for r in rr
                    for j in range(DIM // 16)
                ]
                os_ = [
                    es[t * (DIM // 16) + j] * jnp.float32(SCALE)
                    + sp[j] * ms[t]
                    for t in range(2)
                    for j in range(DIM // 16)
                ]
                for t in range(2):
                    for j in range(DIM // 16):
                        plsc.store_scatter(
                            ob2, [scat[j], rbcs[t]], os_[t * (DIM // 16) + j]
                        )
            return carry

        lax.fori_loop(0, BW // 16, bg_body, 0)

    fire_gather(0, 0)
    fire_gather(1, 1)

    def outer(k, carry):
        for j in range(NBUF):
            c = NBUF * k + j

            @pl.when(c + 2 < L)
            def _():
                fire_gather(c + 2, (j + 2) % NBUF)

            wait_gather(j)

            @pl.when(k > 0)
            def _():
                wait_out(j)

            compute(c, j)
            fire_out(c, j)
        return carry

    lax.fori_loop(0, L // NBUF, outer, 0)

    for j in range(NBUF):
        wait_out(j)


@functools.partial(jax.jit, static_argnames=())
def _run(labelsT, table, pos_table):
    mesh = plsc.VectorSubcoreMesh(
        core_axis_name="c", subcore_axis_name="s", num_cores=NC, num_subcores=NS
    )
    f = pl.kernel(
        _sc_body,
        out_type=jax.ShapeDtypeStruct((L * 8 * NW, 8, 128), jnp.float32),
        mesh=mesh,
        compiler_params=pltpu.CompilerParams(
            use_tc_tiling_on_sc=False, needs_layout_passes=False
        ),
        scratch_types=(
            [
                pltpu.VMEM((L, 128), jnp.int32),
                pltpu.VMEM((MAXLEN, DIM), jnp.float32),
            ]
            + [pltpu.VMEM((BW, DIM), jnp.float32) for _ in range(NBUF)]
            + [pltpu.VMEM((64, 129), jnp.float32) for _ in range(NBUF)]
            + [pltpu.SemaphoreType.DMA for _ in range(2 * NBUF)]
        ),
    )
    return f(labelsT, table, pos_table)


def kernel(labels, label_table, pos_table):
    labelsT = labels.astype(jnp.int32).T
    out4 = _run(labelsT, label_table, pos_table)
    out = (
        out4.reshape(L, 8, NW, 8, 128)
        .transpose(2, 4, 0, 1, 3)
        .reshape(B, L, DIM)
    )
    return out


# speedup vs baseline: 2.3462x; 1.0253x over previous
import functools

import jax
import jax.numpy as jnp
from jax import lax
from jax.experimental import pallas as pl
from jax.experimental.pallas import tpu as pltpu
from jax.experimental.pallas import tpu_sc as plsc

B = 4096
L = 200
DIM = 64
MAXLEN = 256
NC = 2
NS = 16
NW = NC * NS
BW = B // NW
NBUF = 4
SCALE = 0.7071067811865476


def _bcast_lane(vec, i):
    idx = jnp.full((16, 1), i, jnp.int32)
    return lax.gather(
        vec,
        idx,
        dimension_numbers=lax.GatherDimensionNumbers(
            offset_dims=(), collapsed_slice_dims=(0,), start_index_map=(0,)
        ),
        slice_sizes=(1,),
        mode=lax.GatherScatterMode.PROMISE_IN_BOUNDS,
    )


def _sc_body(
    labelsT_hbm,
    table_hbm,
    pos_hbm,
    out_hbm,
    labT,
    pos_v,
    e0, e1, e2, e3,
    b0, b1, b2, b3,
    g0, g1, g2, g3,
    o0, o1, o2, o3,
):
    wid = lax.axis_index("s") * NC + lax.axis_index("c")
    ebufs = [e0, e1, e2, e3]
    obufs = [b0, b1, b2, b3]
    gsems = [g0, g1, g2, g3]
    osems = [o0, o1, o2, o3]

    pltpu.sync_copy(labelsT_hbm.at[:, pl.ds(BW * wid, BW)], labT)
    pltpu.sync_copy(pos_hbm, pos_v)

    def fire_gather(c, buf):
        pltpu.async_copy(table_hbm.at[labT.at[c]], ebufs[buf], gsems[buf])

    def wait_gather(buf):
        pltpu.make_async_copy(
            table_hbm.at[labT.at[0]], ebufs[buf], gsems[buf]
        ).wait()

    def fire_out(c, buf):
        for tf in range(8):
            pltpu.async_copy(
                obufs[buf].at[pl.ds(8 * tf, 8), pl.ds(0, 128)],
                out_hbm.at[(8 * c + tf) * NW + wid],
                osems[buf],
            )

    def wait_out(buf):
        for tf in range(8):
            pltpu.make_async_copy(
                obufs[buf].at[pl.ds(8 * tf, 8), pl.ds(0, 128)],
                out_hbm.at[0],
                osems[buf],
            ).wait()

    iota16 = lax.iota(jnp.int32, 16)
    scat = [iota16 + 16 * j for j in range(DIM // 16)]

    def compute(c, buf):
        E = ebufs[buf]
        ob2 = obufs[buf]
        sp = [
            pos_v[c + 1, pl.ds(16 * j, 16)] * jnp.float32(SCALE)
            for j in range(DIM // 16)
        ]

        def bg_body(bg, carry):
            lab16 = labT[c, pl.ds(16 * bg, 16)]
            m16 = lax.sign(lab16).astype(jnp.float32)
            for i in range(0, 16, 4):
                rr = [16 * bg + i + t for t in range(4)]
                ms = [_bcast_lane(m16, i + t) for t in range(4)]
                rbcs = [jnp.full((16,), 1, jnp.int32) * r for r in rr]
                es = [
                    E[r, pl.ds(16 * j, 16)]
                    for r in rr
                    for j in range(DIM // 16)
                ]
                os_ = [
                    es[t * (DIM // 16) + j] * jnp.float32(SCALE)
                    + sp[j] * ms[t]
                    for t in range(4)
                    for j in range(DIM // 16)
                ]
                for t in range(4):
                    for j in range(DIM // 16):
                        plsc.store_scatter(
                            ob2, [scat[j], rbcs[t]], os_[t * (DIM // 16) + j]
                        )
            return carry

        lax.fori_loop(0, BW // 16, bg_body, 0)

    fire_gather(0, 0)
    fire_gather(1, 1)

    def outer(k, carry):
        for j in range(NBUF):
            c = NBUF * k + j

            @pl.when(c + 2 < L)
            def _():
                fire_gather(c + 2, (j + 2) % NBUF)

            wait_gather(j)

            @pl.when(k > 0)
            def _():
                wait_out(j)

            compute(c, j)
            fire_out(c, j)
        return carry

    lax.fori_loop(0, L // NBUF, outer, 0)

    for j in range(NBUF):
        wait_out(j)


@functools.partial(jax.jit, static_argnames=())
def _run(labelsT, table, pos_table):
    mesh = plsc.VectorSubcoreMesh(
        core_axis_name="c", subcore_axis_name="s", num_cores=NC, num_subcores=NS
    )
    f = pl.kernel(
        _sc_body,
        out_type=jax.ShapeDtypeStruct((L * 8 * NW, 8, 128), jnp.float32),
        mesh=mesh,
        compiler_params=pltpu.CompilerParams(
            use_tc_tiling_on_sc=False, needs_layout_passes=False
        ),
        scratch_types=(
            [
                pltpu.VMEM((L, 128), jnp.int32),
                pltpu.VMEM((MAXLEN, DIM), jnp.float32),
            ]
            + [pltpu.VMEM((BW, DIM), jnp.float32) for _ in range(NBUF)]
            + [pltpu.VMEM((64, 129), jnp.float32) for _ in range(NBUF)]
            + [pltpu.SemaphoreType.DMA for _ in range(2 * NBUF)]
        ),
    )
    return f(labelsT, table, pos_table)


def kernel(labels, label_table, pos_table):
    labelsT = labels.astype(jnp.int32).T
    out4 = _run(labelsT, label_table, pos_table)
    out = (
        out4.reshape(L, 8, NW, 8, 128)
        .transpose(2, 4, 0, 1, 3)
        .reshape(B, L, DIM)
    )
    return out
